# baseline (speedup 1.0000x reference)
import functools
import math

import jax
import jax.numpy as jnp
from jax import lax
from jax.experimental import pallas as pl
from jax.experimental.pallas import tpu as pltpu
from jax.experimental.pallas import tpu_sc as plsc

D_MODEL = 64
VOCAB = 1000000
B, L = 16384, 50
B_TOTAL = B * L
SCALE = math.sqrt(D_MODEL)

NC, NS, LANES = 2, 16, 16
NW = NC * NS
B_PER_W = B // NW
NB = 8
ROWS = NB * L
N_CHUNKS = B_PER_W // NB
G_CHUNKS = NW * N_CHUNKS
NSUB = 5
SUB = ROWS // NSUB


def _emb_body(x2_hbm, par_hbm, lut_hbm, out_hbm, i2_v, par_v, g_v, o_v, sem):
    wid = lax.axis_index("s") * NC + lax.axis_index("c")

    def chunk_body(ci, carry):
        g = wid * N_CHUNKS + ci
        off = wid * (B_PER_W * L) + ci * ROWS
        b0 = wid * B_PER_W + ci * NB
        pltpu.sync_copy(x2_hbm.at[g], i2_v)
        pltpu.sync_copy(par_hbm.at[pl.ds(off, ROWS)], par_v.at[pl.ds(0, ROWS)])
        handles = [
            pltpu.async_copy(
                lut_hbm.at[i2_v.at[j]], g_v.at[pl.ds(j * SUB, SUB), :], sem
            )
            for j in range(NSUB)
        ]
        for h in handles:
            h.wait()

        def b_body(b_l, c2):
            for lg in range(4):
                n_l = min(LANES, L - lg * LANES)
                i0 = b_l * L + lg * LANES
                pv = par_v[pl.ds(i0, LANES)]
                for k in range(n_l):
                    base = pv[k] * D_MODEL
                    i = i0 + k
                    l = lg * LANES + k
                    for j in range(D_MODEL // LANES):
                        o_v[b_l, l, pl.ds(j * LANES, LANES)] = (
                            g_v[i, pl.ds(base + j * LANES, LANES)] * SCALE
                        )
            return c2

        lax.fori_loop(0, NB, b_body, 0)
        pltpu.sync_copy(o_v, out_hbm.at[pl.ds(b0, NB)])
        return carry

    lax.fori_loop(0, N_CHUNKS, chunk_body, 0)


_emb = functools.partial(
    pl.kernel,
    mesh=plsc.VectorSubcoreMesh(core_axis_name="c", subcore_axis_name="s"),
    out_type=jax.ShapeDtypeStruct((B, L, D_MODEL), jnp.float32),
    scratch_types=[
        pltpu.VMEM((NSUB, SUB), jnp.int32),
        pltpu.VMEM((ROWS + LANES,), jnp.int32),
        pltpu.VMEM((ROWS, 2 * D_MODEL), jnp.float32),
        pltpu.VMEM((NB, L, D_MODEL), jnp.float32),
        pltpu.SemaphoreType.DMA,
    ],
    compiler_params=pltpu.CompilerParams(use_tc_tiling_on_sc=True),
)(_emb_body)


def kernel(x, lut):
    xf = x.reshape(B_TOTAL)
    idx2 = (xf >> 1).reshape(G_CHUNKS, NSUB, SUB)
    par = xf & 1
    return _emb(idx2, par, lut.reshape(VOCAB // 2, 2 * D_MODEL))


# speedup vs baseline: 1.0726x; 1.0726x over previous
import functools
import math

import jax
import jax.numpy as jnp
from jax import lax
from jax.experimental import pallas as pl
from jax.experimental.pallas import tpu as pltpu
from jax.experimental.pallas import tpu_sc as plsc

D_MODEL = 64
VOCAB = 1000000
B, L = 16384, 50
B_TOTAL = B * L
SCALE = math.sqrt(D_MODEL)

NC, NS, LANES = 2, 16, 16
NW = NC * NS
B_PER_W = B // NW
NB = 8
ROWS = NB * L
N_CHUNKS = B_PER_W // NB
G_CHUNKS = NW * N_CHUNKS
NSUB = 5
SUB = ROWS // NSUB


def _emb_body(x2_hbm, lut_hbm, out_hbm, i2_v, g_v, o_v, sem):
    wid = lax.axis_index("s") * NC + lax.axis_index("c")

    def chunk_body(ci, carry):
        g = wid * N_CHUNKS + ci
        b0 = wid * B_PER_W + ci * NB
        pltpu.sync_copy(x2_hbm.at[g], i2_v)
        handles = [
            pltpu.async_copy(
                lut_hbm.at[i2_v.at[j]], g_v.at[pl.ds(j * SUB, SUB), :], sem
            )
            for j in range(NSUB)
        ]
        for h in handles:
            h.wait()

        def b_body(b_l, c2):
            for l in range(L):
                i = b_l * L + l
                for j in range(D_MODEL // LANES):
                    o_v[b_l, l, pl.ds(j * LANES, LANES)] = (
                        g_v[i, pl.ds(j * LANES, LANES)] * SCALE
                    )
            return c2

        lax.fori_loop(0, NB, b_body, 0)
        pltpu.sync_copy(o_v, out_hbm.at[pl.ds(b0, NB)])
        return carry

    lax.fori_loop(0, N_CHUNKS, chunk_body, 0)


_emb = functools.partial(
    pl.kernel,
    mesh=plsc.VectorSubcoreMesh(core_axis_name="c", subcore_axis_name="s"),
    out_type=jax.ShapeDtypeStruct((B, L, D_MODEL), jnp.float32),
    scratch_types=[
        pltpu.VMEM((NSUB, SUB), jnp.int32),
        pltpu.VMEM((ROWS, 2 * D_MODEL), jnp.float32),
        pltpu.VMEM((NB, L, D_MODEL), jnp.float32),
        pltpu.SemaphoreType.DMA,
    ],
    compiler_params=pltpu.CompilerParams(use_tc_tiling_on_sc=True),
)(_emb_body)


def kernel(x, lut):
    idx = x.reshape(G_CHUNKS, NSUB, SUB)
    lutp = jnp.pad(lut, ((0, 0), (0, D_MODEL)))
    return _emb(idx, lutp)
